# baseline (speedup 1.0000x reference)
import functools

import jax
import jax.numpy as jnp
from jax import lax
from jax.experimental import pallas as pl
from jax.experimental.pallas import tpu as pltpu
from jax.experimental.pallas import tpu_sc as plsc

N = 10000
E = 160000
EP = 161280
NA = N + 8
NC = 2
NS = 16
RPS = N // NS

_MESH = dict(core_axis_name="c", subcore_axis_name="s")


_DEG_B = 80
_DEG_EPW = EP // (NC * NS)
_DEG_NB = _DEG_EPW // _DEG_B


@functools.partial(
    pl.kernel,
    out_type=jax.ShapeDtypeStruct((NC, NS, RPS, 128), jnp.float32),
    mesh=plsc.VectorSubcoreMesh(**_MESH),
    scratch_types=[
        pltpu.VMEM((_DEG_NB, _DEG_B), jnp.int32),
        pltpu.VMEM((_DEG_B, 128), jnp.float32),
        pltpu.VMEM_SHARED((NA, 128), jnp.float32),
        pltpu.SemaphoreType.DMA,
    ],
)
def _deg_kernel(dst_hbm, z_hbm, ones_hbm, out_hbm, idx_v, ones_v, acc_sh, sem):
    c = lax.axis_index("c")
    s = lax.axis_index("s")
    pltpu.sync_copy(z_hbm, acc_sh.at[pl.ds(s * RPS, RPS)])
    pltpu.sync_copy(ones_hbm, ones_v)
    pltpu.sync_copy(dst_hbm.at[c, s], idx_v)
    plsc.subcore_barrier()

    def fire(j, carry):
        pltpu.async_copy(ones_v, acc_sh.at[idx_v.at[j]], sem, add=True)
        return carry

    lax.fori_loop(0, _DEG_NB, fire, 0)

    def drain(j, carry):
        pltpu.make_async_copy(ones_hbm, ones_v, sem).wait()
        return carry

    lax.fori_loop(0, _DEG_NB, drain, 0)
    plsc.subcore_barrier()
    pltpu.sync_copy(acc_sh.at[pl.ds(s * RPS, RPS)], out_hbm.at[c, s])


_AGG_B = 80
_AGG_EPS = E // NS
_AGG_NB = _AGG_EPS // _AGG_B


def _agg_body(h_hbm, src_i, dst_i, rows0, rows1, g0, g1, s0, s1,
              acc_sh, nb, b):
    def g(j):
        return h_hbm.at[src_i.at[pl.ds(j * b, b)]]

    pltpu.async_copy(g(0), rows0, g0)

    def pair(j2, carry):
        b0 = 2 * j2
        pltpu.async_copy(g(b0 + 1), rows1, g1)
        pltpu.make_async_copy(g(0), rows0, g0).wait()
        pltpu.sync_copy(rows0, acc_sh.at[dst_i.at[b0]], add=True)
        pltpu.async_copy(g(b0 + 2), rows0, g0)
        pltpu.make_async_copy(g(0), rows1, g1).wait()
        pltpu.sync_copy(rows1, acc_sh.at[dst_i.at[b0 + 1]], add=True)
        return carry

    lax.fori_loop(0, (nb - 1) // 2, pair, 0)
    pltpu.make_async_copy(g(0), rows0, g0).wait()
    pltpu.sync_copy(rows0, acc_sh.at[dst_i.at[nb - 1]], add=True)


@functools.partial(
    pl.kernel,
    out_type=jax.ShapeDtypeStruct((NC, NS, RPS, 128), jnp.float32),
    mesh=plsc.VectorSubcoreMesh(**_MESH),
    scratch_types=[
        pltpu.VMEM((_AGG_EPS,), jnp.int32),
        pltpu.VMEM((_AGG_NB, _AGG_B), jnp.int32),
        pltpu.VMEM((_AGG_B, 128), jnp.float32),
        pltpu.VMEM((_AGG_B, 128), jnp.float32),
        pltpu.VMEM_SHARED((N, 128), jnp.float32),
        pltpu.SemaphoreType.DMA,
        pltpu.SemaphoreType.DMA,
        pltpu.SemaphoreType.DMA,
        pltpu.SemaphoreType.DMA,
    ],
)
def _agg_split(h2n_hbm, srcc_hbm, dst_hbm, z_hbm, out_hbm,
               src_i, dst_i, rows0, rows1, acc_sh, g0, g1, s0, s1):
    c = lax.axis_index("c")
    s = lax.axis_index("s")
    pltpu.sync_copy(z_hbm, acc_sh.at[pl.ds(s * RPS, RPS)])
    pltpu.sync_copy(srcc_hbm.at[c, s], src_i)
    pltpu.sync_copy(dst_hbm.at[s], dst_i)
    plsc.subcore_barrier()
    _agg_body(h2n_hbm, src_i, dst_i, rows0, rows1, g0, g1, s0, s1, acc_sh,
              _AGG_NB, _AGG_B)
    plsc.subcore_barrier()
    pltpu.sync_copy(acc_sh.at[pl.ds(s * RPS, RPS)], out_hbm.at[c, s])

_AGG2_B = 80
_AGG2_EPW = EP // (NC * NS)
_AGG2_NB = _AGG2_EPW // _AGG2_B


@functools.partial(
    pl.kernel,
    out_type=jax.ShapeDtypeStruct((NC, NS, RPS, 128), jnp.float32),
    mesh=plsc.VectorSubcoreMesh(**_MESH),
    scratch_types=[
        pltpu.VMEM((_AGG2_EPW,), jnp.int32),
        pltpu.VMEM((_AGG2_NB, _AGG2_B), jnp.int32),
        pltpu.VMEM((_AGG2_B, 128), jnp.float32),
        pltpu.VMEM((_AGG2_B, 128), jnp.float32),
        pltpu.VMEM_SHARED((NA, 128), jnp.float32),
        pltpu.SemaphoreType.DMA,
        pltpu.SemaphoreType.DMA,
        pltpu.SemaphoreType.DMA,
        pltpu.SemaphoreType.DMA,
    ],
)
def _agg_full(h_hbm, src_hbm, dst_hbm, z_hbm, out_hbm,
              src_i, dst_i, rows0, rows1, acc_sh, g0, g1, s0, s1):
    c = lax.axis_index("c")
    s = lax.axis_index("s")
    pltpu.sync_copy(z_hbm, acc_sh.at[pl.ds(s * RPS, RPS)])
    pltpu.sync_copy(src_hbm.at[c, s], src_i)
    pltpu.sync_copy(dst_hbm.at[c, s], dst_i)
    plsc.subcore_barrier()
    _agg_body(h_hbm, src_i, dst_i, rows0, rows1, g0, g1, s0, s1, acc_sh,
              _AGG2_NB, _AGG2_B)
    plsc.subcore_barrier()
    pltpu.sync_copy(acc_sh.at[pl.ds(s * RPS, RPS)], out_hbm.at[c, s])


_BM = 1000
_GRID = N // _BM


def _dinv_from(p):
    return lax.rsqrt(1.0 + p[0, :, 0] + p[1, :, 0])


def _tc_a1_body(x_ref, w1_ref, u_ref):
    u_ref[...] = jnp.dot(x_ref[...], w1_ref[...],
                         preferred_element_type=jnp.float32,
                         precision=lax.Precision.HIGHEST)


_tc_a1 = pl.pallas_call(
    _tc_a1_body,
    grid=(_GRID,),
    in_specs=[
        pl.BlockSpec((_BM, 256), lambda m: (m, 0)),
        pl.BlockSpec((256, 256), lambda m: (0, 0)),
    ],
    out_specs=pl.BlockSpec((_BM, 256), lambda m: (m, 0)),
    out_shape=jax.ShapeDtypeStruct((N, 256), jnp.float32),
)


def _tc_a2_body(u_ref, p_ref, out_ref, d8_ref):
    dinv = _dinv_from(p_ref[...])
    hs = u_ref[...] * dinv[:, None]
    out_ref[0] = hs[:, :128]
    out_ref[1] = hs[:, 128:]
    d8_ref[...] = jnp.broadcast_to(dinv[:, None], (dinv.shape[0], 8))


_tc_a2 = pl.pallas_call(
    _tc_a2_body,
    grid=(_GRID,),
    in_specs=[
        pl.BlockSpec((_BM, 256), lambda m: (m, 0)),
        pl.BlockSpec((2, _BM, 128), lambda m: (0, m, 0)),
    ],
    out_specs=(pl.BlockSpec((2, _BM, 128), lambda m: (0, m, 0)),
               pl.BlockSpec((_BM, 8), lambda m: (m, 0))),
    out_shape=(jax.ShapeDtypeStruct((2, N, 128), jnp.float32),
               jax.ShapeDtypeStruct((N, 8), jnp.float32)),
)


def _tc_b_body(a1_ref, h1s_ref, d8_ref, b1_ref, w2_ref, h2s_ref):
    dinv = d8_ref[:, 0]
    t = jnp.concatenate(
        [a1_ref[0] + h1s_ref[0], a1_ref[1] + h1s_ref[1]], axis=1)
    t = jnp.maximum(t * dinv[:, None] + b1_ref[...], 0.0)
    h2 = jnp.dot(t, w2_ref[...], preferred_element_type=jnp.float32,
                 precision=lax.Precision.HIGHEST)
    h2s_ref[...] = h2 * dinv[:, None]


_tc_b = pl.pallas_call(
    _tc_b_body,
    grid=(_GRID,),
    in_specs=[
        pl.BlockSpec((2, _BM, 128), lambda m: (0, m, 0)),
        pl.BlockSpec((2, _BM, 128), lambda m: (0, m, 0)),
        pl.BlockSpec((_BM, 8), lambda m: (m, 0)),
        pl.BlockSpec((1, 256), lambda m: (0, 0)),
        pl.BlockSpec((256, 128), lambda m: (0, 0)),
    ],
    out_specs=pl.BlockSpec((_BM, 128), lambda m: (m, 0)),
    out_shape=jax.ShapeDtypeStruct((N, 128), jnp.float32),
)


def _tc_c_body(a2_ref, h2s_ref, d8_ref, b2_ref, out_ref):
    dinv = d8_ref[:, 0]
    t = a2_ref[0] + a2_ref[1] + h2s_ref[...]
    out_ref[...] = jnp.maximum(t * dinv[:, None] + b2_ref[...], 0.0)


_tc_c = pl.pallas_call(
    _tc_c_body,
    grid=(_GRID,),
    in_specs=[
        pl.BlockSpec((2, _BM, 128), lambda m: (0, m, 0)),
        pl.BlockSpec((_BM, 128), lambda m: (m, 0)),
        pl.BlockSpec((_BM, 8), lambda m: (m, 0)),
        pl.BlockSpec((1, 128), lambda m: (0, 0)),
    ],
    out_specs=pl.BlockSpec((_BM, 128), lambda m: (m, 0)),
    out_shape=jax.ShapeDtypeStruct((N, 128), jnp.float32),
)


def kernel(x, edge_index, W1, b1, W2, b2):
    src = edge_index[0].astype(jnp.int32)
    dst = edge_index[1].astype(jnp.int32)
    z128 = jnp.zeros((RPS, 128), jnp.float32)
    ones128 = jnp.ones((_DEG_B, 128), jnp.float32)
    srcp = jnp.concatenate([src, jnp.zeros((EP - E,), jnp.int32)])
    dstp = jnp.concatenate([dst, jnp.full((EP - E,), N, jnp.int32)])
    dst4 = dstp.reshape(NC, NS, _AGG2_NB, _AGG2_B)
    src3 = srcp.reshape(NC, NS, _AGG2_EPW)
    dst3 = dst.reshape(NS, _AGG_NB, _AGG_B)
    srcc = jnp.stack([src, src + N]).reshape(NC, NS, _AGG_EPS)

    u = _tc_a1(x, W1)
    p = _deg_kernel(dst4, z128, ones128).reshape(NC, N, 128)
    h1s, d8 = _tc_a2(u, p)
    a1 = _agg_split(h1s.reshape(2 * N, 128), srcc, dst3, z128)
    h2s = _tc_b(a1.reshape(NC, N, 128), h1s, d8, b1.reshape(1, 256), W2)
    a2 = _agg_full(h2s, src3, dst4, z128).reshape(NC, N, 128)
    return _tc_c(a2, h2s, d8, b2.reshape(1, 128))


# speedup vs baseline: 1.1039x; 1.1039x over previous
import functools

import jax
import jax.numpy as jnp
from jax import lax
from jax.experimental import pallas as pl
from jax.experimental.pallas import tpu as pltpu
from jax.experimental.pallas import tpu_sc as plsc

N = 10000
E = 160000
NC = 2
NS = 16
RPS = N // NS

_MESH = dict(core_axis_name="c", subcore_axis_name="s")


_DEG_B = 40
_DEG_EPW = E // (NC * NS)
_DEG_NB = _DEG_EPW // _DEG_B


@functools.partial(
    pl.kernel,
    out_type=jax.ShapeDtypeStruct((NC, NS, RPS, 128), jnp.float32),
    mesh=plsc.VectorSubcoreMesh(**_MESH),
    scratch_types=[
        pltpu.VMEM((_DEG_NB, _DEG_B), jnp.int32),
        pltpu.VMEM((_DEG_B, 128), jnp.float32),
        pltpu.VMEM_SHARED((N, 128), jnp.float32),
        pltpu.SemaphoreType.DMA,
    ],
)
def _deg_kernel(dst_hbm, z_hbm, ones_hbm, out_hbm, idx_v, ones_v, acc_sh, sem):
    c = lax.axis_index("c")
    s = lax.axis_index("s")
    pltpu.sync_copy(z_hbm, acc_sh.at[pl.ds(s * RPS, RPS)])
    pltpu.sync_copy(ones_hbm, ones_v)
    pltpu.sync_copy(dst_hbm.at[c, s], idx_v)
    plsc.subcore_barrier()

    def fire(j, carry):
        pltpu.async_copy(ones_v, acc_sh.at[idx_v.at[j]], sem, add=True)
        return carry

    lax.fori_loop(0, _DEG_NB, fire, 0)

    def drain(j, carry):
        pltpu.make_async_copy(ones_hbm, ones_v, sem).wait()
        return carry

    lax.fori_loop(0, _DEG_NB, drain, 0)
    plsc.subcore_barrier()
    pltpu.sync_copy(acc_sh.at[pl.ds(s * RPS, RPS)], out_hbm.at[c, s])


_AGG_B = 80
_AGG_EPS = E // NS
_AGG_NB = _AGG_EPS // _AGG_B


def _agg_body(h_hbm, src_i, dst_i, rows0, rows1, g0, g1, s0, s1,
              acc_sh, nb, b):
    def g(j):
        return h_hbm.at[src_i.at[pl.ds(j * b, b)]]

    pltpu.async_copy(g(0), rows0, g0)

    def pair(j2, carry):
        b0 = 2 * j2
        pltpu.async_copy(g(b0 + 1), rows1, g1)
        pltpu.make_async_copy(g(0), rows0, g0).wait()
        pltpu.sync_copy(rows0, acc_sh.at[dst_i.at[b0]], add=True)
        pltpu.async_copy(g(b0 + 2), rows0, g0)
        pltpu.make_async_copy(g(0), rows1, g1).wait()
        pltpu.sync_copy(rows1, acc_sh.at[dst_i.at[b0 + 1]], add=True)
        return carry

    lax.fori_loop(0, (nb - 1) // 2, pair, 0)
    pltpu.make_async_copy(g(0), rows0, g0).wait()
    pltpu.sync_copy(rows0, acc_sh.at[dst_i.at[nb - 1]], add=True)


@functools.partial(
    pl.kernel,
    out_type=jax.ShapeDtypeStruct((NC, NS, RPS, 128), jnp.float32),
    mesh=plsc.VectorSubcoreMesh(**_MESH),
    scratch_types=[
        pltpu.VMEM((_AGG_EPS,), jnp.int32),
        pltpu.VMEM((_AGG_NB, _AGG_B), jnp.int32),
        pltpu.VMEM((_AGG_B, 128), jnp.float32),
        pltpu.VMEM((_AGG_B, 128), jnp.float32),
        pltpu.VMEM_SHARED((N, 128), jnp.float32),
        pltpu.SemaphoreType.DMA,
        pltpu.SemaphoreType.DMA,
        pltpu.SemaphoreType.DMA,
        pltpu.SemaphoreType.DMA,
    ],
)
def _agg_split(h2n_hbm, srcc_hbm, dst_hbm, z_hbm, out_hbm,
               src_i, dst_i, rows0, rows1, acc_sh, g0, g1, s0, s1):
    c = lax.axis_index("c")
    s = lax.axis_index("s")
    pltpu.sync_copy(z_hbm, acc_sh.at[pl.ds(s * RPS, RPS)])
    pltpu.sync_copy(srcc_hbm.at[c, s], src_i)
    pltpu.sync_copy(dst_hbm.at[s], dst_i)
    plsc.subcore_barrier()
    _agg_body(h2n_hbm, src_i, dst_i, rows0, rows1, g0, g1, s0, s1, acc_sh,
              _AGG_NB, _AGG_B)
    plsc.subcore_barrier()
    pltpu.sync_copy(acc_sh.at[pl.ds(s * RPS, RPS)], out_hbm.at[c, s])

_AGG2_B = 40
_AGG2_EPW = E // (NC * NS)
_AGG2_NB = _AGG2_EPW // _AGG2_B


@functools.partial(
    pl.kernel,
    out_type=jax.ShapeDtypeStruct((NC, NS, RPS, 128), jnp.float32),
    mesh=plsc.VectorSubcoreMesh(**_MESH),
    scratch_types=[
        pltpu.VMEM((_AGG2_EPW,), jnp.int32),
        pltpu.VMEM((_AGG2_NB, _AGG2_B), jnp.int32),
        pltpu.VMEM((_AGG2_B, 128), jnp.float32),
        pltpu.VMEM((_AGG2_B, 128), jnp.float32),
        pltpu.VMEM_SHARED((N, 128), jnp.float32),
        pltpu.SemaphoreType.DMA,
        pltpu.SemaphoreType.DMA,
        pltpu.SemaphoreType.DMA,
        pltpu.SemaphoreType.DMA,
    ],
)
def _agg_full(h_hbm, src_hbm, dst_hbm, z_hbm, out_hbm,
              src_i, dst_i, rows0, rows1, acc_sh, g0, g1, s0, s1):
    c = lax.axis_index("c")
    s = lax.axis_index("s")
    pltpu.sync_copy(z_hbm, acc_sh.at[pl.ds(s * RPS, RPS)])
    pltpu.sync_copy(src_hbm.at[c, s], src_i)
    pltpu.sync_copy(dst_hbm.at[c, s], dst_i)
    plsc.subcore_barrier()
    _agg_body(h_hbm, src_i, dst_i, rows0, rows1, g0, g1, s0, s1, acc_sh,
              _AGG2_NB, _AGG2_B)
    plsc.subcore_barrier()
    pltpu.sync_copy(acc_sh.at[pl.ds(s * RPS, RPS)], out_hbm.at[c, s])


_BM = 1000
_GRID = N // _BM


def _dinv_from(p):
    return lax.rsqrt(1.0 + p[0, :, 0] + p[1, :, 0])


def _tc_a1_body(x_ref, w1_ref, u_ref):
    u_ref[...] = jnp.dot(x_ref[...], w1_ref[...],
                         preferred_element_type=jnp.float32,
                         precision=lax.Precision.HIGHEST)


_tc_a1 = pl.pallas_call(
    _tc_a1_body,
    grid=(_GRID,),
    in_specs=[
        pl.BlockSpec((_BM, 256), lambda m: (m, 0)),
        pl.BlockSpec((256, 256), lambda m: (0, 0)),
    ],
    out_specs=pl.BlockSpec((_BM, 256), lambda m: (m, 0)),
    out_shape=jax.ShapeDtypeStruct((N, 256), jnp.float32),
)


def _tc_a2_body(u_ref, p_ref, out_ref, d8_ref):
    dinv = _dinv_from(p_ref[...])
    hs = u_ref[...] * dinv[:, None]
    out_ref[0] = hs[:, :128]
    out_ref[1] = hs[:, 128:]
    d8_ref[...] = jnp.broadcast_to(dinv[:, None], (dinv.shape[0], 8))


_tc_a2 = pl.pallas_call(
    _tc_a2_body,
    grid=(_GRID,),
    in_specs=[
        pl.BlockSpec((_BM, 256), lambda m: (m, 0)),
        pl.BlockSpec((2, _BM, 128), lambda m: (0, m, 0)),
    ],
    out_specs=(pl.BlockSpec((2, _BM, 128), lambda m: (0, m, 0)),
               pl.BlockSpec((_BM, 8), lambda m: (m, 0))),
    out_shape=(jax.ShapeDtypeStruct((2, N, 128), jnp.float32),
               jax.ShapeDtypeStruct((N, 8), jnp.float32)),
)


def _tc_b_body(a1_ref, h1s_ref, d8_ref, b1_ref, w2_ref, h2s_ref):
    dinv = d8_ref[:, 0]
    t = jnp.concatenate(
        [a1_ref[0] + h1s_ref[0], a1_ref[1] + h1s_ref[1]], axis=1)
    t = jnp.maximum(t * dinv[:, None] + b1_ref[...], 0.0)
    h2 = jnp.dot(t, w2_ref[...], preferred_element_type=jnp.float32,
                 precision=lax.Precision.HIGHEST)
    h2s_ref[...] = h2 * dinv[:, None]


_tc_b = pl.pallas_call(
    _tc_b_body,
    grid=(_GRID,),
    in_specs=[
        pl.BlockSpec((2, _BM, 128), lambda m: (0, m, 0)),
        pl.BlockSpec((2, _BM, 128), lambda m: (0, m, 0)),
        pl.BlockSpec((_BM, 8), lambda m: (m, 0)),
        pl.BlockSpec((1, 256), lambda m: (0, 0)),
        pl.BlockSpec((256, 128), lambda m: (0, 0)),
    ],
    out_specs=pl.BlockSpec((_BM, 128), lambda m: (m, 0)),
    out_shape=jax.ShapeDtypeStruct((N, 128), jnp.float32),
)


def _tc_c_body(a2_ref, h2s_ref, d8_ref, b2_ref, out_ref):
    dinv = d8_ref[:, 0]
    t = a2_ref[0] + a2_ref[1] + h2s_ref[...]
    out_ref[...] = jnp.maximum(t * dinv[:, None] + b2_ref[...], 0.0)


_tc_c = pl.pallas_call(
    _tc_c_body,
    grid=(_GRID,),
    in_specs=[
        pl.BlockSpec((2, _BM, 128), lambda m: (0, m, 0)),
        pl.BlockSpec((_BM, 128), lambda m: (m, 0)),
        pl.BlockSpec((_BM, 8), lambda m: (m, 0)),
        pl.BlockSpec((1, 128), lambda m: (0, 0)),
    ],
    out_specs=pl.BlockSpec((_BM, 128), lambda m: (m, 0)),
    out_shape=jax.ShapeDtypeStruct((N, 128), jnp.float32),
)


def kernel(x, edge_index, W1, b1, W2, b2):
    src = edge_index[0].astype(jnp.int32)
    dst = edge_index[1].astype(jnp.int32)
    z128 = jnp.zeros((RPS, 128), jnp.float32)
    ones128 = jnp.ones((_DEG_B, 128), jnp.float32)
    dst4 = dst.reshape(NC, NS, _AGG2_NB, _AGG2_B)
    src3 = src.reshape(NC, NS, _AGG2_EPW)
    dst3 = dst.reshape(NS, _AGG_NB, _AGG_B)
    srcc = jnp.stack([src, src + N]).reshape(NC, NS, _AGG_EPS)

    u = _tc_a1(x, W1)
    p = _deg_kernel(dst4, z128, ones128).reshape(NC, N, 128)
    h1s, d8 = _tc_a2(u, p)
    a1 = _agg_split(h1s.reshape(2 * N, 128), srcc, dst3, z128)
    h2s = _tc_b(a1.reshape(NC, N, 128), h1s, d8, b1.reshape(1, 256), W2)
    a2 = _agg_full(h2s, src3, dst4, z128).reshape(NC, N, 128)
    return _tc_c(a2, h2s, d8, b2.reshape(1, 128))
